# baseline (speedup 1.0000x reference)
import dataclasses

import jax
import jax.numpy as jnp
from jax import lax
from jax.experimental import pallas as pl
from jax.experimental.pallas import tpu as pltpu
from jax.experimental.pallas import tpu_sc as plsc

_BATCH = 4096
_NCARDS = 26
_CARD = 100
_WIDTH = _NCARDS * _CARD
_NC, _NS = 2, 16
_NW = _NC * _NS
_ROWS_W = _BATCH // _NW
_BLK = 32
_NBLK = _ROWS_W // _BLK
_IDX_W = _ROWS_W * _NCARDS
_IDX_BLK = _BLK * _NCARDS
_NVEC = _IDX_BLK // 16


def _scatter_block(buf, xbuf, blk, val):
    for v in range(_NVEC):
        base = blk * _IDX_BLK + v * 16
        p = base + lax.iota(jnp.int32, 16)
        xv = xbuf[pl.ds(base, 16)]
        row = p // _NCARDS - blk * _BLK
        col = (p % _NCARDS) * _CARD + xv
        plsc.store_scatter(buf, [row, col], val)


def _sc_onehot(zeros_hbm, idx_hbm, out_hbm, buf, xbuf, sem):
    wid = lax.axis_index("s") * _NC + lax.axis_index("c")
    row0 = wid * _ROWS_W
    ones = jnp.full((16,), 1, jnp.int32)
    zeros = jnp.zeros((16,), jnp.int32)

    pltpu.async_copy(zeros_hbm, buf, sem).wait()
    pltpu.sync_copy(idx_hbm.at[pl.ds(row0 * _NCARDS, _IDX_W)], xbuf)

    for blk in range(_NBLK):
        _scatter_block(buf, xbuf, blk - 1 if blk else 0, zeros)
        _scatter_block(buf, xbuf, blk, ones)
        pltpu.sync_copy(buf, out_hbm.at[pl.ds(row0 + blk * _BLK, _BLK)])


def kernel(x):
    idx = x.reshape(_BATCH * _NCARDS)
    zeros2d = jnp.zeros((_BLK, _WIDTH), jnp.int32)
    mesh = plsc.VectorSubcoreMesh(core_axis_name="c", subcore_axis_name="s")
    cp = pltpu.CompilerParams()
    if "needs_layout_passes" in pltpu.CompilerParams.__dataclass_fields__:
        cp = dataclasses.replace(cp, needs_layout_passes=False)
    run = pl.kernel(
        _sc_onehot,
        out_type=jax.ShapeDtypeStruct((_BATCH, _WIDTH), jnp.int32),
        mesh=mesh,
        scratch_types=[
            pltpu.VMEM((_BLK, _WIDTH), jnp.int32),
            pltpu.VMEM((_IDX_W,), jnp.int32),
            pltpu.SemaphoreType.DMA,
        ],
        compiler_params=cp,
    )
    return run(zeros2d, idx)


# speedup vs baseline: 1.0353x; 1.0353x over previous
import dataclasses

import jax
import jax.numpy as jnp
from jax import lax
from jax.experimental import pallas as pl
from jax.experimental.pallas import tpu as pltpu
from jax.experimental.pallas import tpu_sc as plsc

_BATCH = 4096
_NCARDS = 26
_CARD = 100
_WIDTH = _NCARDS * _CARD
_NC, _NS = 2, 16
_NW = _NC * _NS
_ROWS_W = _BATCH // _NW
_BLK = 16
_NBLK = _ROWS_W // _BLK
_IDX_BLK = _BLK * _NCARDS
_NVEC = _IDX_BLK // 16


def _scatter_block(buf, xbuf, val):
    for v in range(_NVEC):
        p = v * 16 + lax.iota(jnp.int32, 16)
        xv = xbuf[pl.ds(v * 16, 16)]
        row = p // _NCARDS
        col = (p % _NCARDS) * _CARD + xv
        plsc.store_scatter(buf, [row, col], val)


def _sc_onehot(zeros_hbm, idx_hbm, out_hbm, buf, xbuf, sem):
    wid = lax.axis_index("s") * _NC + lax.axis_index("c")
    row0 = wid * _ROWS_W

    ones = jnp.full((16,), 1, jnp.int32)
    zeros = jnp.zeros((16,), jnp.int32)

    pltpu.async_copy(zeros_hbm, buf, sem).wait()
    pltpu.sync_copy(idx_hbm.at[pl.ds(row0 * _NCARDS, _IDX_BLK)], xbuf)

    @pl.loop(0, _NBLK)
    def _(blk):
        _scatter_block(buf, xbuf, zeros)
        pltpu.sync_copy(
            idx_hbm.at[pl.ds((row0 + blk * _BLK) * _NCARDS, _IDX_BLK)], xbuf)
        _scatter_block(buf, xbuf, ones)
        pltpu.sync_copy(buf, out_hbm.at[pl.ds(row0 + blk * _BLK, _BLK)])


def kernel(x):
    idx = x.reshape(_BATCH * _NCARDS)
    zeros2d = jnp.zeros((_BLK, _WIDTH), jnp.int32)
    mesh = plsc.VectorSubcoreMesh(core_axis_name="c", subcore_axis_name="s")
    cp = pltpu.CompilerParams()
    if "needs_layout_passes" in pltpu.CompilerParams.__dataclass_fields__:
        cp = dataclasses.replace(cp, needs_layout_passes=False)
    run = pl.kernel(
        _sc_onehot,
        out_type=jax.ShapeDtypeStruct((_BATCH, _WIDTH), jnp.int32),
        mesh=mesh,
        scratch_types=[
            pltpu.VMEM((_BLK, _WIDTH), jnp.int32),
            pltpu.VMEM((_IDX_BLK,), jnp.int32),
            pltpu.SemaphoreType.DMA,
        ],
        compiler_params=cp,
    )
    return run(zeros2d, idx)
